# baseline (device time: 149831 ns/iter reference)
import jax
import jax.numpy as jnp
from jax import lax
from jax.experimental import pallas as pl
from jax.experimental.pallas import tpu as pltpu

N_DEV = 16


def kernel(A, B):
    m, k = A.shape
    _, n = B.shape
    ch = m // N_DEV

    def body(a_ref, b_ref, out_ref, z_ref, rs_ref, ag_ref,
             rs_send, rs_recv, ag_send, ag_recv):
        my = lax.axis_index("i")
        left = (my - 1) % N_DEV
        right = (my + 1) % N_DEV

        barrier_sem = pltpu.get_barrier_semaphore()
        for nbr in (left, right):
            pl.semaphore_signal(barrier_sem, inc=1, device_id=(nbr,),
                                device_id_type=pl.DeviceIdType.MESH)
        pl.semaphore_wait(barrier_sem, 2)

        z_ref[...] = jnp.dot(a_ref[...].astype(jnp.bfloat16),
                             b_ref[...].astype(jnp.bfloat16),
                             preferred_element_type=jnp.float32)

        rs_ref[0] = z_ref[pl.ds(my * ch, ch), :]
        for s in range(N_DEV - 1):
            rdma = pltpu.make_async_remote_copy(
                src_ref=rs_ref.at[s],
                dst_ref=rs_ref.at[s + 1],
                send_sem=rs_send.at[s],
                recv_sem=rs_recv.at[s],
                device_id=(right,),
                device_id_type=pl.DeviceIdType.MESH,
            )
            rdma.start()
            rdma.wait()
            c = (my - s - 1) % N_DEV
            rs_ref[s + 1] = rs_ref[s + 1] + z_ref[pl.ds(c * ch, ch), :]

        z = rs_ref[N_DEV - 1]
        g = 0.5 * z * (1.0 + jnp.tanh(0.7978845608 * (z + 0.044715 * z * z * z)))
        out_ref[pl.ds(((my + 1) % N_DEV) * ch, ch), :] = g
        ag_ref[0] = g

        for s in range(N_DEV - 1):
            rdma = pltpu.make_async_remote_copy(
                src_ref=ag_ref.at[s],
                dst_ref=ag_ref.at[s + 1],
                send_sem=ag_send.at[s],
                recv_sem=ag_recv.at[s],
                device_id=(right,),
                device_id_type=pl.DeviceIdType.MESH,
            )
            rdma.start()
            rdma.wait()
            c = (my - s) % N_DEV
            out_ref[pl.ds(c * ch, ch), :] = ag_ref[s + 1]

    return pl.pallas_call(
        body,
        out_shape=jax.ShapeDtypeStruct((m, n), jnp.float32),
        in_specs=[pl.BlockSpec(memory_space=pltpu.VMEM),
                  pl.BlockSpec(memory_space=pltpu.VMEM)],
        out_specs=pl.BlockSpec(memory_space=pltpu.VMEM),
        scratch_shapes=[
            pltpu.VMEM((m, n), jnp.float32),
            pltpu.VMEM((N_DEV, ch, n), jnp.float32),
            pltpu.VMEM((N_DEV, ch, n), jnp.float32),
            pltpu.SemaphoreType.DMA((N_DEV - 1,)),
            pltpu.SemaphoreType.DMA((N_DEV - 1,)),
            pltpu.SemaphoreType.DMA((N_DEV - 1,)),
            pltpu.SemaphoreType.DMA((N_DEV - 1,)),
        ],
        compiler_params=pltpu.CompilerParams(collective_id=0),
    )(A, B)


# device time: 68032 ns/iter; 2.2024x vs baseline; 2.2024x over previous
import jax
import jax.numpy as jnp
from jax import lax
from jax.experimental import pallas as pl
from jax.experimental.pallas import tpu as pltpu

N_DEV = 16
RS_SIZES = (512, 256, 128, 64)


def _gelu(z):
    return 0.5 * z * (1.0 + jnp.tanh(0.7978845608 * (z + 0.044715 * z * z * z)))


def kernel(A, B):
    m, k = A.shape
    _, n = B.shape

    def body(a_ref, b_ref, out_ref, z_ref, ag_ref,
             s0, s1, s2, s3, r0, r1, r2, r3,
             rs_ssem, rs_rsem, ag_ssem, ag_rsem):
        my = lax.axis_index("i")
        zc = my >> 2
        p = my & 3

        a_bits = [(p ^ (p >> 1)) & 1, p >> 1, zc & 1, zc >> 1]
        partners = [
            (zc << 2) | (p ^ 1),
            (zc << 2) | (p ^ 3),
            ((zc ^ 1) << 2) | p,
            ((zc ^ 2) << 2) | p,
        ]

        barrier_sem = pltpu.get_barrier_semaphore()
        for nbr in partners:
            pl.semaphore_signal(barrier_sem, inc=1, device_id=(nbr,),
                                device_id_type=pl.DeviceIdType.MESH)
        pl.semaphore_wait(barrier_sem, len(partners))

        b_bf = b_ref[...].astype(jnp.bfloat16)
        sbufs = [s0, s1, s2, s3]
        rbufs = [r0, r1, r2, r3]
        rdmas = []

        start = my * 0
        for kk in range(4):
            half = RS_SIZES[kk]
            bit = a_bits[kk]
            send_start = start + (1 - bit) * half
            keep_start = start + bit * half
            if kk == 0:
                z_ref[pl.ds(send_start, half), :] = jnp.dot(
                    a_ref[pl.ds(send_start, half), :].astype(jnp.bfloat16),
                    b_bf, preferred_element_type=jnp.float32)
            sbufs[kk][...] = z_ref[pl.ds(send_start, half), :].astype(jnp.bfloat16)
            rdma = pltpu.make_async_remote_copy(
                src_ref=sbufs[kk],
                dst_ref=rbufs[kk],
                send_sem=rs_ssem.at[kk],
                recv_sem=rs_rsem.at[kk],
                device_id=(partners[kk],),
                device_id_type=pl.DeviceIdType.MESH,
            )
            rdma.start()
            rdmas.append(rdma)
            if kk == 0:
                z_ref[pl.ds(keep_start, half), :] = jnp.dot(
                    a_ref[pl.ds(keep_start, half), :].astype(jnp.bfloat16),
                    b_bf, preferred_element_type=jnp.float32)
            rdma.wait_recv()
            z_ref[pl.ds(keep_start, half), :] = (
                z_ref[pl.ds(keep_start, half), :]
                + rbufs[kk][...].astype(jnp.float32))
            start = keep_start

        g = _gelu(z_ref[pl.ds(start, 64), :])
        out_ref[pl.ds(start, 64), :] = g
        cstart = start >> 6
        ag_ref[pl.ds(cstart, 1)] = g.astype(jnp.bfloat16)[None]

        base = cstart
        pending = None
        for kk in range(4):
            nch = 1 << kk
            partner_base = base ^ nch
            rdma = pltpu.make_async_remote_copy(
                src_ref=ag_ref.at[pl.ds(base, nch)],
                dst_ref=ag_ref.at[pl.ds(base, nch)],
                send_sem=ag_ssem.at[kk],
                recv_sem=ag_rsem.at[kk],
                device_id=(partners[3 - kk],),
                device_id_type=pl.DeviceIdType.MESH,
            )
            rdma.start()
            rdmas.append(rdma)
            if pending is not None:
                cb, cs = pending
                out_ref[pl.ds(cb * 64, cs * 64), :] = (
                    ag_ref[pl.ds(cb, cs)].reshape(cs * 64, n).astype(jnp.float32))
            rdma.wait_recv()
            pending = (partner_base, nch)
            base = base & ~nch
        cb, cs = pending
        out_ref[pl.ds(cb * 64, cs * 64), :] = (
            ag_ref[pl.ds(cb, cs)].reshape(cs * 64, n).astype(jnp.float32))

        for rdma in rdmas:
            rdma.wait_send()

    return pl.pallas_call(
        body,
        out_shape=jax.ShapeDtypeStruct((m, n), jnp.float32),
        in_specs=[pl.BlockSpec(memory_space=pltpu.VMEM),
                  pl.BlockSpec(memory_space=pltpu.VMEM)],
        out_specs=pl.BlockSpec(memory_space=pltpu.VMEM),
        scratch_shapes=[
            pltpu.VMEM((m, n), jnp.float32),
            pltpu.VMEM((16, 64, n), jnp.bfloat16),
            pltpu.VMEM((512, n), jnp.bfloat16),
            pltpu.VMEM((256, n), jnp.bfloat16),
            pltpu.VMEM((128, n), jnp.bfloat16),
            pltpu.VMEM((64, n), jnp.bfloat16),
            pltpu.VMEM((512, n), jnp.bfloat16),
            pltpu.VMEM((256, n), jnp.bfloat16),
            pltpu.VMEM((128, n), jnp.bfloat16),
            pltpu.VMEM((64, n), jnp.bfloat16),
            pltpu.SemaphoreType.DMA((4,)),
            pltpu.SemaphoreType.DMA((4,)),
            pltpu.SemaphoreType.DMA((4,)),
            pltpu.SemaphoreType.DMA((4,)),
        ],
        compiler_params=pltpu.CompilerParams(collective_id=0),
    )(A, B)


# device time: 67074 ns/iter; 2.2338x vs baseline; 1.0143x over previous
import jax
import jax.numpy as jnp
from jax import lax
from jax.experimental import pallas as pl
from jax.experimental.pallas import tpu as pltpu

N_DEV = 16
RS_SIZES = (512, 256, 128, 64)


def _gelu(z):
    return 0.5 * z * (1.0 + jnp.tanh(0.7978845608 * (z + 0.044715 * z * z * z)))


def kernel(A, B):
    m, k = A.shape
    _, n = B.shape

    def body(a_ref, b_ref, out_ref, z_ref,
             s0, s1, s2, s3, r0, r1, r2, r3,
             rs_ssem, rs_rsem, ag_ssem, ag_rsem):
        my = lax.axis_index("i")
        zc = my >> 2
        p = my & 3

        a_bits = [(p ^ (p >> 1)) & 1, p >> 1, zc & 1, zc >> 1]
        partners = [
            (zc << 2) | (p ^ 1),
            (zc << 2) | (p ^ 3),
            ((zc ^ 1) << 2) | p,
            ((zc ^ 2) << 2) | p,
        ]

        barrier_sem = pltpu.get_barrier_semaphore()
        for nbr in partners:
            pl.semaphore_signal(barrier_sem, inc=1, device_id=(nbr,),
                                device_id_type=pl.DeviceIdType.MESH)
        pl.semaphore_wait(barrier_sem, len(partners))

        b_bf = b_ref[...].astype(jnp.bfloat16)
        sbufs = [s0, s1, s2, s3]
        rbufs = [r0, r1, r2, r3]

        def exchange(kk, sems_pair):
            rdma = pltpu.make_async_remote_copy(
                src_ref=sbufs[kk],
                dst_ref=rbufs[kk],
                send_sem=sems_pair[0].at[kk],
                recv_sem=sems_pair[1].at[kk],
                device_id=(partners[kk],),
                device_id_type=pl.DeviceIdType.MESH,
            )
            rdma.start()
            return rdma

        rdmas = []

        bit = a_bits[0]
        send_start = (1 - bit) * 512
        start = bit * 512
        z_ref[pl.ds(send_start, 512), :] = jnp.dot(
            a_ref[pl.ds(send_start, 512), :].astype(jnp.bfloat16),
            b_bf, preferred_element_type=jnp.float32)
        sbufs[0][...] = z_ref[pl.ds(send_start, 512), :].astype(jnp.bfloat16)
        rdmas.append(exchange(0, (rs_ssem, rs_rsem)))
        z_ref[pl.ds(start, 512), :] = jnp.dot(
            a_ref[pl.ds(start, 512), :].astype(jnp.bfloat16),
            b_bf, preferred_element_type=jnp.float32)

        for kk in range(1, 4):
            half = RS_SIZES[kk]
            bit = a_bits[kk]
            rel_s = (1 - bit) * half
            rel_k = bit * half
            rdmas[kk - 1].wait_recv()
            z_ref[pl.ds(start + rel_s, half), :] = (
                z_ref[pl.ds(start + rel_s, half), :]
                + rbufs[kk - 1][pl.ds(rel_s, half), :].astype(jnp.float32))
            sbufs[kk][...] = z_ref[pl.ds(start + rel_s, half), :].astype(jnp.bfloat16)
            rdmas.append(exchange(kk, (rs_ssem, rs_rsem)))
            z_ref[pl.ds(start + rel_k, half), :] = (
                z_ref[pl.ds(start + rel_k, half), :]
                + rbufs[kk - 1][pl.ds(rel_k, half), :].astype(jnp.float32))
            start = start + rel_k

        rdmas[3].wait_recv()
        zz = z_ref[pl.ds(start, 64), :] + rbufs[3][...].astype(jnp.float32)
        g = _gelu(zz)
        cstart = start >> 6
        out_ref[pl.ds(cstart, 1)] = g.astype(jnp.bfloat16)[None]

        base = cstart
        for kk in range(4):
            nch = 1 << kk
            rdma = pltpu.make_async_remote_copy(
                src_ref=out_ref.at[pl.ds(base, nch)],
                dst_ref=out_ref.at[pl.ds(base, nch)],
                send_sem=ag_ssem.at[kk],
                recv_sem=ag_rsem.at[kk],
                device_id=(partners[3 - kk],),
                device_id_type=pl.DeviceIdType.MESH,
            )
            rdma.start()
            rdmas.append(rdma)
            rdma.wait_recv()
            base = base & ~nch

        for rdma in rdmas:
            rdma.wait_send()

    out = pl.pallas_call(
        body,
        out_shape=jax.ShapeDtypeStruct((16, m // 16, n), jnp.bfloat16),
        in_specs=[pl.BlockSpec(memory_space=pltpu.VMEM),
                  pl.BlockSpec(memory_space=pltpu.VMEM)],
        out_specs=pl.BlockSpec(memory_space=pltpu.VMEM),
        scratch_shapes=[
            pltpu.VMEM((m, n), jnp.float32),
            pltpu.VMEM((512, n), jnp.bfloat16),
            pltpu.VMEM((256, n), jnp.bfloat16),
            pltpu.VMEM((128, n), jnp.bfloat16),
            pltpu.VMEM((64, n), jnp.bfloat16),
            pltpu.VMEM((512, n), jnp.bfloat16),
            pltpu.VMEM((256, n), jnp.bfloat16),
            pltpu.VMEM((128, n), jnp.bfloat16),
            pltpu.VMEM((64, n), jnp.bfloat16),
            pltpu.SemaphoreType.DMA((4,)),
            pltpu.SemaphoreType.DMA((4,)),
            pltpu.SemaphoreType.DMA((4,)),
            pltpu.SemaphoreType.DMA((4,)),
        ],
        compiler_params=pltpu.CompilerParams(collective_id=0),
    )(A, B)
    return out.reshape(m, n)


# device time: 48591 ns/iter; 3.0835x vs baseline; 1.3804x over previous
import jax
import jax.numpy as jnp
from jax import lax
from jax.experimental import pallas as pl
from jax.experimental.pallas import tpu as pltpu

N_DEV = 16
SIZES = (256, 128, 64, 32)


def _gelu(z):
    return 0.5 * z * (1.0 + jnp.tanh(0.7978845608 * (z + 0.044715 * z * z * z)))


def kernel(A, B):
    m, k = A.shape
    _, n = B.shape

    def body(a_ref, b_ref, out_ref, z_ref,
             sa0, sa1, sa2, sa3, ra0, ra1, ra2, ra3,
             sb0, sb1, sb2, sb3, rb0, rb1, rb2, rb3,
             rsa_ssem, rsa_rsem, rsb_ssem, rsb_rsem,
             aga_ssem, aga_rsem, agb_ssem, agb_rsem):
        my = lax.axis_index("i")
        zc = my >> 2
        p = my & 3

        bx = (p ^ (p >> 1)) & 1
        by = p >> 1
        bz0 = zc & 1
        bz1 = zc >> 1
        px = (zc << 2) | (p ^ 1)
        py = (zc << 2) | (p ^ 3)
        pz0 = ((zc ^ 1) << 2) | p
        pz1 = ((zc ^ 2) << 2) | p

        bits_a, parts_a = [bx, by, bz0, bz1], [px, py, pz0, pz1]
        bits_b, parts_b = [by, bx, bz1, bz0], [py, px, pz1, pz0]

        barrier_sem = pltpu.get_barrier_semaphore()
        for nbr in parts_a:
            pl.semaphore_signal(barrier_sem, inc=1, device_id=(nbr,),
                                device_id_type=pl.DeviceIdType.MESH)
        pl.semaphore_wait(barrier_sem, 4)

        b_bf = b_ref[...].astype(jnp.bfloat16)
        sa = [sa0, sa1, sa2, sa3]
        ra = [ra0, ra1, ra2, ra3]
        sb = [sb0, sb1, sb2, sb3]
        rb = [rb0, rb1, rb2, rb3]
        rdmas = []

        def exchange(src, dst, ssem, rsem, kk, partner):
            rdma = pltpu.make_async_remote_copy(
                src_ref=src, dst_ref=dst,
                send_sem=ssem.at[kk], recv_sem=rsem.at[kk],
                device_id=(partner,), device_id_type=pl.DeviceIdType.MESH,
            )
            rdma.start()
            rdmas.append(rdma)
            return rdma

        send_a = (1 - bits_a[0]) * 256
        start_a = bits_a[0] * 256
        send_b = 512 + (1 - bits_b[0]) * 256
        start_b = 512 + bits_b[0] * 256
        for s in (send_a, send_b):
            z_ref[pl.ds(s, 256), :] = jnp.dot(
                a_ref[pl.ds(s, 256), :].astype(jnp.bfloat16),
                b_bf, preferred_element_type=jnp.float32)
        sa[0][...] = z_ref[pl.ds(send_a, 256), :].astype(jnp.bfloat16)
        da = exchange(sa[0], ra[0], rsa_ssem, rsa_rsem, 0, parts_a[0])
        sb[0][...] = z_ref[pl.ds(send_b, 256), :].astype(jnp.bfloat16)
        db = exchange(sb[0], rb[0], rsb_ssem, rsb_rsem, 0, parts_b[0])
        for s in (start_a, start_b):
            z_ref[pl.ds(s, 256), :] = jnp.dot(
                a_ref[pl.ds(s, 256), :].astype(jnp.bfloat16),
                b_bf, preferred_element_type=jnp.float32)

        for kk in range(1, 4):
            half = SIZES[kk]
            ra_s = (1 - bits_a[kk]) * half
            ra_k = bits_a[kk] * half
            rb_s = (1 - bits_b[kk]) * half
            rb_k = bits_b[kk] * half

            da.wait_recv()
            z_ref[pl.ds(start_a + ra_s, half), :] = (
                z_ref[pl.ds(start_a + ra_s, half), :]
                + ra[kk - 1][pl.ds(ra_s, half), :].astype(jnp.float32))
            sa[kk][...] = z_ref[pl.ds(start_a + ra_s, half), :].astype(jnp.bfloat16)
            da = exchange(sa[kk], ra[kk], rsa_ssem, rsa_rsem, kk, parts_a[kk])

            db.wait_recv()
            z_ref[pl.ds(start_b + rb_s, half), :] = (
                z_ref[pl.ds(start_b + rb_s, half), :]
                + rb[kk - 1][pl.ds(rb_s, half), :].astype(jnp.float32))
            sb[kk][...] = z_ref[pl.ds(start_b + rb_s, half), :].astype(jnp.bfloat16)
            db = exchange(sb[kk], rb[kk], rsb_ssem, rsb_rsem, kk, parts_b[kk])

            z_ref[pl.ds(start_a + ra_k, half), :] = (
                z_ref[pl.ds(start_a + ra_k, half), :]
                + ra[kk - 1][pl.ds(ra_k, half), :].astype(jnp.float32))
            z_ref[pl.ds(start_b + rb_k, half), :] = (
                z_ref[pl.ds(start_b + rb_k, half), :]
                + rb[kk - 1][pl.ds(rb_k, half), :].astype(jnp.float32))
            start_a = start_a + ra_k
            start_b = start_b + rb_k

        da.wait_recv()
        ga = _gelu(z_ref[pl.ds(start_a, 32), :] + ra[3][...].astype(jnp.float32))
        ca = start_a >> 5
        out_ref[pl.ds(ca, 1)] = ga.astype(jnp.bfloat16)[None]
        db.wait_recv()
        gb = _gelu(z_ref[pl.ds(start_b, 32), :] + rb[3][...].astype(jnp.float32))
        cb = start_b >> 5
        out_ref[pl.ds(cb, 1)] = gb.astype(jnp.bfloat16)[None]

        base_a, base_b = ca, cb
        for kk in range(4):
            nch = 1 << kk
            da = exchange(out_ref.at[pl.ds(base_a, nch)],
                          out_ref.at[pl.ds(base_a, nch)],
                          aga_ssem, aga_rsem, kk, parts_a[3 - kk])
            db = exchange(out_ref.at[pl.ds(base_b, nch)],
                          out_ref.at[pl.ds(base_b, nch)],
                          agb_ssem, agb_rsem, kk, parts_b[3 - kk])
            da.wait_recv()
            db.wait_recv()
            base_a = base_a & ~nch
            base_b = base_b & ~nch

        for rdma in rdmas:
            rdma.wait_send()

    out = pl.pallas_call(
        body,
        out_shape=jax.ShapeDtypeStruct((32, m // 32, n), jnp.bfloat16),
        in_specs=[pl.BlockSpec(memory_space=pltpu.VMEM),
                  pl.BlockSpec(memory_space=pltpu.VMEM)],
        out_specs=pl.BlockSpec(memory_space=pltpu.VMEM),
        scratch_shapes=[
            pltpu.VMEM((m, n), jnp.float32),
            pltpu.VMEM((256, n), jnp.bfloat16),
            pltpu.VMEM((128, n), jnp.bfloat16),
            pltpu.VMEM((64, n), jnp.bfloat16),
            pltpu.VMEM((32, n), jnp.bfloat16),
            pltpu.VMEM((256, n), jnp.bfloat16),
            pltpu.VMEM((128, n), jnp.bfloat16),
            pltpu.VMEM((64, n), jnp.bfloat16),
            pltpu.VMEM((32, n), jnp.bfloat16),
            pltpu.VMEM((256, n), jnp.bfloat16),
            pltpu.VMEM((128, n), jnp.bfloat16),
            pltpu.VMEM((64, n), jnp.bfloat16),
            pltpu.VMEM((32, n), jnp.bfloat16),
            pltpu.VMEM((256, n), jnp.bfloat16),
            pltpu.VMEM((128, n), jnp.bfloat16),
            pltpu.VMEM((64, n), jnp.bfloat16),
            pltpu.VMEM((32, n), jnp.bfloat16),
            pltpu.SemaphoreType.DMA((4,)),
            pltpu.SemaphoreType.DMA((4,)),
            pltpu.SemaphoreType.DMA((4,)),
            pltpu.SemaphoreType.DMA((4,)),
            pltpu.SemaphoreType.DMA((4,)),
            pltpu.SemaphoreType.DMA((4,)),
            pltpu.SemaphoreType.DMA((4,)),
            pltpu.SemaphoreType.DMA((4,)),
        ],
        compiler_params=pltpu.CompilerParams(collective_id=0),
    )(A, B)
    return out.reshape(m, n)


# device time: 48375 ns/iter; 3.0973x vs baseline; 1.0045x over previous
import jax
import jax.numpy as jnp
from jax import lax
from jax.experimental import pallas as pl
from jax.experimental.pallas import tpu as pltpu

N_DEV = 16
SIZES = (256, 128, 64, 32)


def _gelu(z):
    return 0.5 * z * (1.0 + jnp.tanh(0.7978845608 * (z + 0.044715 * z * z * z)))


def kernel(A, B):
    m, k = A.shape
    _, n = B.shape

    def body(a_ref, b_ref, out_ref, z_ref,
             sa0, sa1, sa2, sa3, ra0, ra1, ra2, ra3,
             sb0, sb1, sb2, sb3, rb0, rb1, rb2, rb3,
             rsa_ssem, rsa_rsem, rsb_ssem, rsb_rsem,
             aga_ssem, aga_rsem, agb_ssem, agb_rsem):
        my = lax.axis_index("i")
        zc = my >> 2
        p = my & 3

        bx = (p ^ (p >> 1)) & 1
        by = p >> 1
        bz0 = zc & 1
        bz1 = zc >> 1
        px = (zc << 2) | (p ^ 1)
        py = (zc << 2) | (p ^ 3)
        pz0 = ((zc ^ 1) << 2) | p
        pz1 = ((zc ^ 2) << 2) | p

        bits_a, parts_a = [bx, by, bz0, bz1], [px, py, pz0, pz1]
        bits_b, parts_b = [by, bx, bz1, bz0], [py, px, pz1, pz0]

        barrier_sem = pltpu.get_barrier_semaphore()
        for nbr in parts_a:
            pl.semaphore_signal(barrier_sem, inc=1, device_id=(nbr,),
                                device_id_type=pl.DeviceIdType.MESH)

        b_bf = b_ref[...].astype(jnp.bfloat16)
        sa = [sa0, sa1, sa2, sa3]
        ra = [ra0, ra1, ra2, ra3]
        sb = [sb0, sb1, sb2, sb3]
        rb = [rb0, rb1, rb2, rb3]
        rdmas = []

        def exchange(src, dst, ssem, rsem, kk, partner):
            rdma = pltpu.make_async_remote_copy(
                src_ref=src, dst_ref=dst,
                send_sem=ssem.at[kk], recv_sem=rsem.at[kk],
                device_id=(partner,), device_id_type=pl.DeviceIdType.MESH,
            )
            rdma.start()
            rdmas.append(rdma)
            return rdma

        send_a = (1 - bits_a[0]) * 256
        start_a = bits_a[0] * 256
        send_b = 512 + (1 - bits_b[0]) * 256
        start_b = 512 + bits_b[0] * 256
        for s in (send_a, send_b):
            z_ref[pl.ds(s, 256), :] = jnp.dot(
                a_ref[pl.ds(s, 256), :].astype(jnp.bfloat16),
                b_bf, preferred_element_type=jnp.float32)
        sa[0][...] = z_ref[pl.ds(send_a, 256), :].astype(jnp.bfloat16)
        pl.semaphore_wait(barrier_sem, 4)
        da = exchange(sa[0], ra[0], rsa_ssem, rsa_rsem, 0, parts_a[0])
        sb[0][...] = z_ref[pl.ds(send_b, 256), :].astype(jnp.bfloat16)
        db = exchange(sb[0], rb[0], rsb_ssem, rsb_rsem, 0, parts_b[0])
        for s in (start_a, start_b):
            z_ref[pl.ds(s, 256), :] = jnp.dot(
                a_ref[pl.ds(s, 256), :].astype(jnp.bfloat16),
                b_bf, preferred_element_type=jnp.float32)

        for kk in range(1, 4):
            half = SIZES[kk]
            ra_s = (1 - bits_a[kk]) * half
            ra_k = bits_a[kk] * half
            rb_s = (1 - bits_b[kk]) * half
            rb_k = bits_b[kk] * half

            def fwd_a(kk=kk, half=half, ra_s=ra_s, start=start_a):
                da.wait_recv()
                z_ref[pl.ds(start + ra_s, half), :] = (
                    z_ref[pl.ds(start + ra_s, half), :]
                    + ra[kk - 1][pl.ds(ra_s, half), :].astype(jnp.float32))
                sa[kk][...] = z_ref[pl.ds(start + ra_s, half), :].astype(jnp.bfloat16)
                return exchange(sa[kk], ra[kk], rsa_ssem, rsa_rsem, kk, parts_a[kk])

            def fwd_b(kk=kk, half=half, rb_s=rb_s, start=start_b):
                db.wait_recv()
                z_ref[pl.ds(start + rb_s, half), :] = (
                    z_ref[pl.ds(start + rb_s, half), :]
                    + rb[kk - 1][pl.ds(rb_s, half), :].astype(jnp.float32))
                sb[kk][...] = z_ref[pl.ds(start + rb_s, half), :].astype(jnp.bfloat16)
                return exchange(sb[kk], rb[kk], rsb_ssem, rsb_rsem, kk, parts_b[kk])

            if kk % 2 == 1:
                da = fwd_a()
                db = fwd_b()
            else:
                db = fwd_b()
                da = fwd_a()

            z_ref[pl.ds(start_a + ra_k, half), :] = (
                z_ref[pl.ds(start_a + ra_k, half), :]
                + ra[kk - 1][pl.ds(ra_k, half), :].astype(jnp.float32))
            z_ref[pl.ds(start_b + rb_k, half), :] = (
                z_ref[pl.ds(start_b + rb_k, half), :]
                + rb[kk - 1][pl.ds(rb_k, half), :].astype(jnp.float32))
            start_a = start_a + ra_k
            start_b = start_b + rb_k

        da.wait_recv()
        ga = _gelu(z_ref[pl.ds(start_a, 32), :] + ra[3][...].astype(jnp.float32))
        ca = start_a >> 5
        out_ref[pl.ds(ca, 1)] = ga.astype(jnp.bfloat16)[None]
        db.wait_recv()
        gb = _gelu(z_ref[pl.ds(start_b, 32), :] + rb[3][...].astype(jnp.float32))
        cb = start_b >> 5
        out_ref[pl.ds(cb, 1)] = gb.astype(jnp.bfloat16)[None]

        base_a, base_b = ca, cb
        for kk in range(4):
            nch = 1 << kk
            da = exchange(out_ref.at[pl.ds(base_a, nch)],
                          out_ref.at[pl.ds(base_a, nch)],
                          aga_ssem, aga_rsem, kk, parts_a[3 - kk])
            db = exchange(out_ref.at[pl.ds(base_b, nch)],
                          out_ref.at[pl.ds(base_b, nch)],
                          agb_ssem, agb_rsem, kk, parts_b[3 - kk])
            da.wait_recv()
            db.wait_recv()
            base_a = base_a & ~nch
            base_b = base_b & ~nch

        for rdma in rdmas:
            rdma.wait_send()

    out = pl.pallas_call(
        body,
        out_shape=jax.ShapeDtypeStruct((32, m // 32, n), jnp.bfloat16),
        in_specs=[pl.BlockSpec(memory_space=pltpu.VMEM),
                  pl.BlockSpec(memory_space=pltpu.VMEM)],
        out_specs=pl.BlockSpec(memory_space=pltpu.VMEM),
        scratch_shapes=[
            pltpu.VMEM((m, n), jnp.float32),
            pltpu.VMEM((256, n), jnp.bfloat16),
            pltpu.VMEM((128, n), jnp.bfloat16),
            pltpu.VMEM((64, n), jnp.bfloat16),
            pltpu.VMEM((32, n), jnp.bfloat16),
            pltpu.VMEM((256, n), jnp.bfloat16),
            pltpu.VMEM((128, n), jnp.bfloat16),
            pltpu.VMEM((64, n), jnp.bfloat16),
            pltpu.VMEM((32, n), jnp.bfloat16),
            pltpu.VMEM((256, n), jnp.bfloat16),
            pltpu.VMEM((128, n), jnp.bfloat16),
            pltpu.VMEM((64, n), jnp.bfloat16),
            pltpu.VMEM((32, n), jnp.bfloat16),
            pltpu.VMEM((256, n), jnp.bfloat16),
            pltpu.VMEM((128, n), jnp.bfloat16),
            pltpu.VMEM((64, n), jnp.bfloat16),
            pltpu.VMEM((32, n), jnp.bfloat16),
            pltpu.SemaphoreType.DMA((4,)),
            pltpu.SemaphoreType.DMA((4,)),
            pltpu.SemaphoreType.DMA((4,)),
            pltpu.SemaphoreType.DMA((4,)),
            pltpu.SemaphoreType.DMA((4,)),
            pltpu.SemaphoreType.DMA((4,)),
            pltpu.SemaphoreType.DMA((4,)),
            pltpu.SemaphoreType.DMA((4,)),
        ],
        compiler_params=pltpu.CompilerParams(collective_id=0),
    )(A, B)
    return out.reshape(m, n)
